# baseline (speedup 1.0000x reference)
import functools

import jax
import jax.numpy as jnp
from jax import lax
from jax.experimental import pallas as pl
from jax.experimental.pallas import tpu as pltpu
from jax.experimental.pallas import tpu_sc as plsc

L = 16
NC = 2
NS = 16
NW = NC * NS
CHUNK = 128


def _cdiv(a, b):
    return (a + b - 1) // b


def _deg_body(np_pad, cpt, dst_hbm, deg_out, dst_v, ones_v, z_v, deg_sh):
    c = lax.axis_index("c")
    s = lax.axis_index("s")
    wid = c * NS + s
    rpt = np_pad // NS

    def zrow(i, _):
        z_v[pl.ds(i * L, L)] = jnp.zeros((L,), jnp.float32)
        return 0
    lax.fori_loop(0, rpt // L, zrow, 0)
    for k in range(CHUNK // L):
        ones_v[pl.ds(k * L, L)] = jnp.ones((L,), jnp.float32)

    pltpu.sync_copy(z_v, deg_sh.at[pl.ds(s * rpt, rpt)])
    plsc.subcore_barrier()

    pltpu.sync_copy(dst_hbm.at[wid], dst_v)

    def body(i, _):
        pltpu.sync_copy(ones_v, deg_sh.at[dst_v.at[i]], add=True)
        return 0
    lax.fori_loop(0, cpt, body, 0)

    plsc.subcore_barrier()
    pltpu.sync_copy(deg_sh.at[pl.ds(s * rpt, rpt)],
                    deg_out.at[c, pl.ds(s * rpt, rpt)])


def _seg_body(np_pad, cpt0, cpt1, hp_hbm, src_hbm, dst_hbm, acc_out,
              src_v, dst_v, rows_v, acc_sh, sem):
    c = lax.axis_index("c")
    s = lax.axis_index("s")
    rpt = np_pad // NS
    cptmax = max(cpt0, cpt1)
    base = jnp.where(c == 0, s * cpt0, NS * cpt0 + s * cpt1)
    nchunks = jnp.where(c == 0, cpt0, cpt1)

    def zrow(i, _):
        for k in range(8):
            rows_v[i, pl.ds(k * L, L)] = jnp.zeros((L,), jnp.float32)
        return 0
    lax.fori_loop(0, CHUNK, zrow, 0)

    def zcopy(j, _):
        pltpu.sync_copy(rows_v,
                        acc_sh.at[pl.ds(s * rpt + j * CHUNK, CHUNK), :])
        return 0
    lax.fori_loop(0, rpt // CHUNK, zcopy, 0)
    plsc.subcore_barrier()

    pltpu.sync_copy(src_hbm.at[pl.ds(base, cptmax), :], src_v)
    pltpu.sync_copy(dst_hbm.at[pl.ds(base, cptmax), :], dst_v)

    def body(i, _):
        @pl.when(i < nchunks)
        def _():
            pltpu.async_copy(hp_hbm.at[src_v.at[i]], rows_v, sem).wait()
            pltpu.sync_copy(rows_v, acc_sh.at[dst_v.at[i]], add=True)
        return 0
    lax.fori_loop(0, cptmax, body, 0)

    plsc.subcore_barrier()
    pltpu.sync_copy(acc_sh.at[pl.ds(s * rpt, rpt), :],
                    acc_out.at[c, pl.ds(s * rpt, rpt), :])


def _prep_body(x_ref, w1_ref, degt_ref, hp_ref, dis_ref):
    deg = degt_ref[:, 0:1] + degt_ref[:, 1:2] + 1.0
    dis = lax.rsqrt(jnp.maximum(deg, 1e-12))
    h = jnp.dot(x_ref[...], w1_ref[...], preferred_element_type=jnp.float32)
    hp_ref[...] = h * dis
    dis_ref[...] = dis


def _out_body(acc_ref, hp_ref, dis_ref, b1_ref, wmu_ref, wls_ref, epsw_ref,
              bmu_ref, bls_ref, epsb_ref, out_ref):
    t = acc_ref[0] + acc_ref[1] + hp_ref[...]
    pre = t * dis_ref[...] + b1_ref[...]
    hr = jnp.maximum(pre, 0.0)
    w = wmu_ref[...] + jnp.exp(wls_ref[...]) * epsw_ref[...]
    b = bmu_ref[...] + jnp.exp(bls_ref[...]) * epsb_ref[...]
    logits = lax.dot_general(hr, w, (((1,), (1,)), ((), ())),
                             preferred_element_type=jnp.float32) + b
    m = jnp.max(logits, axis=1, keepdims=True)
    ex = jnp.exp(logits - m)
    lse = m + jnp.log(jnp.sum(ex, axis=1, keepdims=True))
    out_ref[...] = logits - lse


def kernel(x, edge_index, W1, b1, w_mu, w_log_sigma, eps_w, b_mu, b_log_sigma,
           eps_b):
    n, d = x.shape
    h = W1.shape[1]
    cls = w_mu.shape[0]
    e = edge_index.shape[1]

    np_pad = _cdiv(n + 1, NS * L) * NS * L
    ep = _cdiv(e, NS * CHUNK * 16) * NS * CHUNK * 16
    cpt = ep // (NW * CHUNK)
    nblk = 10
    rblk = n // nblk
    assert n % nblk == 0 and rblk % 8 == 0

    src = edge_index[0]
    dst = edge_index[1]
    pad_e = ep - e
    srcp = jnp.concatenate([src, jnp.zeros((pad_e,), jnp.int32)])
    dump = n + jnp.arange(pad_e, dtype=jnp.int32) % (np_pad - n)
    dstp = jnp.concatenate([dst, dump])
    dst3 = dstp.reshape(NW, cpt, CHUNK)

    mesh = plsc.VectorSubcoreMesh(core_axis_name="c", subcore_axis_name="s")

    deg_part = pl.kernel(
        functools.partial(_deg_body, np_pad, cpt),
        out_type=jax.ShapeDtypeStruct((NC, np_pad), jnp.float32),
        mesh=mesh,
        scratch_types=[
            pltpu.VMEM((cpt, CHUNK), jnp.int32),
            pltpu.VMEM((CHUNK,), jnp.float32),
            pltpu.VMEM((np_pad // NS,), jnp.float32),
            pltpu.VMEM_SHARED((np_pad,), jnp.float32),
        ],
    )(dst3)

    degt = deg_part.T
    hp, dis = pl.pallas_call(
        _prep_body,
        grid=(nblk,),
        in_specs=[
            pl.BlockSpec((rblk, d), lambda i: (i, 0)),
            pl.BlockSpec((d, h), lambda i: (0, 0)),
            pl.BlockSpec((rblk, NC), lambda i: (i, 0)),
        ],
        out_specs=[
            pl.BlockSpec((rblk, h), lambda i: (i, 0)),
            pl.BlockSpec((rblk, 1), lambda i: (i, 0)),
        ],
        out_shape=[
            jax.ShapeDtypeStruct((n, h), jnp.float32),
            jax.ShapeDtypeStruct((n, 1), jnp.float32),
        ],
    )(x, W1, degt)

    tot_chunks = ep // CHUNK
    cpt_pair = tot_chunks // NS
    cpt0 = max(8, int(round(cpt_pair * 0.35 / 8)) * 8)
    cpt1 = cpt_pair - cpt0
    cptmax = max(cpt0, cpt1)
    src2 = srcp.reshape(tot_chunks, CHUNK)
    dst2 = dstp.reshape(tot_chunks, CHUNK)
    acc_part = pl.kernel(
        functools.partial(_seg_body, np_pad, cpt0, cpt1),
        out_type=jax.ShapeDtypeStruct((NC, np_pad, h), jnp.float32),
        mesh=mesh,
        scratch_types=[
            pltpu.VMEM((cptmax, CHUNK), jnp.int32),
            pltpu.VMEM((cptmax, CHUNK), jnp.int32),
            pltpu.VMEM((CHUNK, h), jnp.float32),
            pltpu.VMEM_SHARED((np_pad, h), jnp.float32),
            pltpu.SemaphoreType.DMA,
        ],
    )(hp, src2, dst2)

    out = pl.pallas_call(
        _out_body,
        grid=(nblk,),
        in_specs=[
            pl.BlockSpec((NC, rblk, h), lambda i: (0, i, 0)),
            pl.BlockSpec((rblk, h), lambda i: (i, 0)),
            pl.BlockSpec((rblk, 1), lambda i: (i, 0)),
            pl.BlockSpec((1, h), lambda i: (0, 0)),
            pl.BlockSpec((cls, h), lambda i: (0, 0)),
            pl.BlockSpec((cls, h), lambda i: (0, 0)),
            pl.BlockSpec((cls, h), lambda i: (0, 0)),
            pl.BlockSpec((1, cls), lambda i: (0, 0)),
            pl.BlockSpec((1, cls), lambda i: (0, 0)),
            pl.BlockSpec((1, cls), lambda i: (0, 0)),
        ],
        out_specs=pl.BlockSpec((rblk, cls), lambda i: (i, 0)),
        out_shape=jax.ShapeDtypeStruct((n, cls), jnp.float32),
    )(acc_part, hp, dis, b1.reshape(1, h), w_mu, w_log_sigma, eps_w,
      b_mu.reshape(1, cls), b_log_sigma.reshape(1, cls),
      eps_b.reshape(1, cls))

    return out


# speedup vs baseline: 1.1538x; 1.1538x over previous
import functools

import jax
import jax.numpy as jnp
from jax import lax
from jax.experimental import pallas as pl
from jax.experimental.pallas import tpu as pltpu
from jax.experimental.pallas import tpu_sc as plsc

L = 16
NC = 2
NS = 16
NW = NC * NS
CHUNK = 128


def _cdiv(a, b):
    return (a + b - 1) // b


def _deg_body(np_pad, cpt, dst_hbm, deg_out, dst_v, ones_v, z_v, deg_sh):
    c = lax.axis_index("c")
    s = lax.axis_index("s")
    wid = c * NS + s
    rpt = np_pad // NS

    def zrow(i, _):
        z_v[pl.ds(i * L, L)] = jnp.zeros((L,), jnp.float32)
        return 0
    lax.fori_loop(0, rpt // L, zrow, 0)
    for k in range(CHUNK // L):
        ones_v[pl.ds(k * L, L)] = jnp.ones((L,), jnp.float32)

    pltpu.sync_copy(z_v, deg_sh.at[pl.ds(s * rpt, rpt)])
    plsc.subcore_barrier()

    pltpu.sync_copy(dst_hbm.at[wid], dst_v)

    def body(i, _):
        pltpu.sync_copy(ones_v, deg_sh.at[dst_v.at[i]], add=True)
        return 0
    lax.fori_loop(0, cpt, body, 0)

    plsc.subcore_barrier()
    pltpu.sync_copy(deg_sh.at[pl.ds(s * rpt, rpt)],
                    deg_out.at[c, pl.ds(s * rpt, rpt)])


def _seg_body(np_pad, cpt0, cpt1, hp_hbm, src_hbm, dst_hbm, acc_out,
              src_v, dst_v, rows_v, acc_sh, sem):
    c = lax.axis_index("c")
    s = lax.axis_index("s")
    rpt = np_pad // NS
    cptmax = max(cpt0, cpt1)
    base = jnp.where(c == 0, s * cpt0, NS * cpt0 + s * cpt1)
    nchunks = jnp.where(c == 0, cpt0, cpt1)

    def zrow(i, _):
        for k in range(8):
            rows_v[i, pl.ds(k * L, L)] = jnp.zeros((L,), jnp.float32)
        return 0
    lax.fori_loop(0, CHUNK, zrow, 0)

    def zcopy(j, _):
        pltpu.sync_copy(rows_v,
                        acc_sh.at[pl.ds(s * rpt + j * CHUNK, CHUNK), :])
        return 0
    lax.fori_loop(0, rpt // CHUNK, zcopy, 0)
    plsc.subcore_barrier()

    pltpu.sync_copy(src_hbm.at[pl.ds(base, cptmax), :], src_v)
    pltpu.sync_copy(dst_hbm.at[pl.ds(base, cptmax), :], dst_v)

    def body(i, _):
        @pl.when(i < nchunks)
        def _():
            pltpu.async_copy(hp_hbm.at[src_v.at[i]], rows_v, sem).wait()
            pltpu.sync_copy(rows_v, acc_sh.at[dst_v.at[i]], add=True)
        return 0
    lax.fori_loop(0, cptmax, body, 0)

    plsc.subcore_barrier()
    pltpu.sync_copy(acc_sh.at[pl.ds(s * rpt, rpt), :],
                    acc_out.at[c, pl.ds(s * rpt, rpt), :])


def _prep_body(x_ref, w1_ref, degt_ref, hp_ref, dis_ref):
    deg = degt_ref[:, 0:1] + degt_ref[:, 1:2] + 1.0
    dis = lax.rsqrt(jnp.maximum(deg, 1e-12))
    h = jnp.dot(x_ref[...], w1_ref[...], preferred_element_type=jnp.float32)
    hp_ref[...] = h * dis
    dis_ref[...] = dis


def _out_body(acc_ref, hp_ref, dis_ref, b1_ref, wmu_ref, wls_ref, epsw_ref,
              bmu_ref, bls_ref, epsb_ref, out_ref):
    t = acc_ref[0] + acc_ref[1] + hp_ref[...]
    pre = t * dis_ref[...] + b1_ref[...]
    hr = jnp.maximum(pre, 0.0)
    w = wmu_ref[...] + jnp.exp(wls_ref[...]) * epsw_ref[...]
    b = bmu_ref[...] + jnp.exp(bls_ref[...]) * epsb_ref[...]
    logits = lax.dot_general(hr, w, (((1,), (1,)), ((), ())),
                             preferred_element_type=jnp.float32) + b
    m = jnp.max(logits, axis=1, keepdims=True)
    ex = jnp.exp(logits - m)
    lse = m + jnp.log(jnp.sum(ex, axis=1, keepdims=True))
    out_ref[...] = logits - lse


def kernel(x, edge_index, W1, b1, w_mu, w_log_sigma, eps_w, b_mu, b_log_sigma,
           eps_b):
    n, d = x.shape
    h = W1.shape[1]
    cls = w_mu.shape[0]
    e = edge_index.shape[1]

    np_pad = _cdiv(n + 1, NS * L) * NS * L
    ep = _cdiv(e, NS * CHUNK * 16) * NS * CHUNK * 16
    cpt = ep // (NW * CHUNK)
    nblk = 10
    rblk = n // nblk
    assert n % nblk == 0 and rblk % 8 == 0

    src = edge_index[0]
    dst = edge_index[1]
    pad_e = ep - e
    srcp = jnp.concatenate([src, jnp.zeros((pad_e,), jnp.int32)])
    dump = n + jnp.arange(pad_e, dtype=jnp.int32) % (np_pad - n)
    dstp = jnp.concatenate([dst, dump])
    dst3 = dstp.reshape(NW, cpt, CHUNK)

    mesh = plsc.VectorSubcoreMesh(core_axis_name="c", subcore_axis_name="s")

    deg_part = pl.kernel(
        functools.partial(_deg_body, np_pad, cpt),
        out_type=jax.ShapeDtypeStruct((NC, np_pad), jnp.float32),
        mesh=mesh,
        scratch_types=[
            pltpu.VMEM((cpt, CHUNK), jnp.int32),
            pltpu.VMEM((CHUNK,), jnp.float32),
            pltpu.VMEM((np_pad // NS,), jnp.float32),
            pltpu.VMEM_SHARED((np_pad,), jnp.float32),
        ],
    )(dst3)

    degt = deg_part.T
    hp, dis = pl.pallas_call(
        _prep_body,
        grid=(nblk,),
        in_specs=[
            pl.BlockSpec((rblk, d), lambda i: (i, 0)),
            pl.BlockSpec((d, h), lambda i: (0, 0)),
            pl.BlockSpec((rblk, NC), lambda i: (i, 0)),
        ],
        out_specs=[
            pl.BlockSpec((rblk, h), lambda i: (i, 0)),
            pl.BlockSpec((rblk, 1), lambda i: (i, 0)),
        ],
        out_shape=[
            jax.ShapeDtypeStruct((n, h), jnp.float32),
            jax.ShapeDtypeStruct((n, 1), jnp.float32),
        ],
    )(x, W1, degt)

    tot_chunks = ep // CHUNK
    cpt_pair = tot_chunks // NS
    cpt1 = max(8, int(round(cpt_pair * 0.35 / 8)) * 8)
    cpt0 = cpt_pair - cpt1
    cptmax = max(cpt0, cpt1)
    src2 = srcp.reshape(tot_chunks, CHUNK)
    dst2 = dstp.reshape(tot_chunks, CHUNK)
    acc_part = pl.kernel(
        functools.partial(_seg_body, np_pad, cpt0, cpt1),
        out_type=jax.ShapeDtypeStruct((NC, np_pad, h), jnp.float32),
        mesh=mesh,
        scratch_types=[
            pltpu.VMEM((cptmax, CHUNK), jnp.int32),
            pltpu.VMEM((cptmax, CHUNK), jnp.int32),
            pltpu.VMEM((CHUNK, h), jnp.float32),
            pltpu.VMEM_SHARED((np_pad, h), jnp.float32),
            pltpu.SemaphoreType.DMA,
        ],
    )(hp, src2, dst2)

    out = pl.pallas_call(
        _out_body,
        grid=(nblk,),
        in_specs=[
            pl.BlockSpec((NC, rblk, h), lambda i: (0, i, 0)),
            pl.BlockSpec((rblk, h), lambda i: (i, 0)),
            pl.BlockSpec((rblk, 1), lambda i: (i, 0)),
            pl.BlockSpec((1, h), lambda i: (0, 0)),
            pl.BlockSpec((cls, h), lambda i: (0, 0)),
            pl.BlockSpec((cls, h), lambda i: (0, 0)),
            pl.BlockSpec((cls, h), lambda i: (0, 0)),
            pl.BlockSpec((1, cls), lambda i: (0, 0)),
            pl.BlockSpec((1, cls), lambda i: (0, 0)),
            pl.BlockSpec((1, cls), lambda i: (0, 0)),
        ],
        out_specs=pl.BlockSpec((rblk, cls), lambda i: (i, 0)),
        out_shape=jax.ShapeDtypeStruct((n, cls), jnp.float32),
    )(acc_part, hp, dis, b1.reshape(1, h), w_mu, w_log_sigma, eps_w,
      b_mu.reshape(1, cls), b_log_sigma.reshape(1, cls),
      eps_b.reshape(1, cls))

    return out


# speedup vs baseline: 1.1544x; 1.0006x over previous
import functools

import jax
import jax.numpy as jnp
from jax import lax
from jax.experimental import pallas as pl
from jax.experimental.pallas import tpu as pltpu
from jax.experimental.pallas import tpu_sc as plsc

L = 16
NC = 2
NS = 16
NW = NC * NS
CHUNK = 128


def _cdiv(a, b):
    return (a + b - 1) // b


def _deg_body(np_pad, cpt, dst_hbm, deg_out, dst_v, ones_v, z_v, deg_sh):
    c = lax.axis_index("c")
    s = lax.axis_index("s")
    wid = c * NS + s
    rpt = np_pad // NS

    def zrow(i, _):
        z_v[pl.ds(i * L, L)] = jnp.zeros((L,), jnp.float32)
        return 0
    lax.fori_loop(0, rpt // L, zrow, 0)
    for k in range(CHUNK // L):
        ones_v[pl.ds(k * L, L)] = jnp.ones((L,), jnp.float32)

    pltpu.sync_copy(z_v, deg_sh.at[pl.ds(s * rpt, rpt)])
    plsc.subcore_barrier()

    pltpu.sync_copy(dst_hbm.at[wid], dst_v)

    def body(i, _):
        pltpu.sync_copy(ones_v, deg_sh.at[dst_v.at[i]], add=True)
        return 0
    lax.fori_loop(0, cpt, body, 0)

    plsc.subcore_barrier()
    pltpu.sync_copy(deg_sh.at[pl.ds(s * rpt, rpt)],
                    deg_out.at[c, pl.ds(s * rpt, rpt)])


def _seg_body(np_pad, cpt0, cpt1, hp_hbm, src_hbm, dst_hbm, acc_out,
              src_v, dst_v, rows_v, acc_sh, sem):
    c = lax.axis_index("c")
    s = lax.axis_index("s")
    rpt = np_pad // NS
    cptmax = max(cpt0, cpt1)
    base = jnp.where(c == 0, s * cpt0, NS * cpt0 + s * cpt1)
    nchunks = jnp.where(c == 0, cpt0, cpt1)

    def zrow(i, _):
        for k in range(8):
            rows_v[i, pl.ds(k * L, L)] = jnp.zeros((L,), jnp.float32)
        return 0
    lax.fori_loop(0, CHUNK, zrow, 0)

    def zcopy(j, _):
        pltpu.sync_copy(rows_v,
                        acc_sh.at[pl.ds(s * rpt + j * CHUNK, CHUNK), :])
        return 0
    lax.fori_loop(0, rpt // CHUNK, zcopy, 0)
    plsc.subcore_barrier()

    pltpu.sync_copy(src_hbm.at[pl.ds(base, cptmax), :], src_v)
    pltpu.sync_copy(dst_hbm.at[pl.ds(base, cptmax), :], dst_v)

    def body(i, _):
        pltpu.async_copy(hp_hbm.at[src_v.at[i]], rows_v, sem).wait()
        pltpu.sync_copy(rows_v, acc_sh.at[dst_v.at[i]], add=True)
        return 0
    lax.fori_loop(0, nchunks, body, 0)

    plsc.subcore_barrier()
    pltpu.sync_copy(acc_sh.at[pl.ds(s * rpt, rpt), :],
                    acc_out.at[c, pl.ds(s * rpt, rpt), :])


def _prep_body(x_ref, w1_ref, degt_ref, hp_ref, dis_ref):
    deg = degt_ref[:, 0:1] + degt_ref[:, 1:2] + 1.0
    dis = lax.rsqrt(jnp.maximum(deg, 1e-12))
    h = jnp.dot(x_ref[...], w1_ref[...], preferred_element_type=jnp.float32)
    hp_ref[...] = h * dis
    dis_ref[...] = dis


def _out_body(acc_ref, hp_ref, dis_ref, b1_ref, wmu_ref, wls_ref, epsw_ref,
              bmu_ref, bls_ref, epsb_ref, out_ref):
    t = acc_ref[0] + acc_ref[1] + hp_ref[...]
    pre = t * dis_ref[...] + b1_ref[...]
    hr = jnp.maximum(pre, 0.0)
    w = wmu_ref[...] + jnp.exp(wls_ref[...]) * epsw_ref[...]
    b = bmu_ref[...] + jnp.exp(bls_ref[...]) * epsb_ref[...]
    logits = lax.dot_general(hr, w, (((1,), (1,)), ((), ())),
                             preferred_element_type=jnp.float32) + b
    m = jnp.max(logits, axis=1, keepdims=True)
    ex = jnp.exp(logits - m)
    lse = m + jnp.log(jnp.sum(ex, axis=1, keepdims=True))
    out_ref[...] = logits - lse


def kernel(x, edge_index, W1, b1, w_mu, w_log_sigma, eps_w, b_mu, b_log_sigma,
           eps_b):
    n, d = x.shape
    h = W1.shape[1]
    cls = w_mu.shape[0]
    e = edge_index.shape[1]

    np_pad = _cdiv(n + 1, NS * L) * NS * L
    ep = _cdiv(e, NS * CHUNK * 16) * NS * CHUNK * 16
    cpt = ep // (NW * CHUNK)
    nblk = 10
    rblk = n // nblk
    assert n % nblk == 0 and rblk % 8 == 0

    src = edge_index[0]
    dst = edge_index[1]
    pad_e = ep - e
    srcp = jnp.concatenate([src, jnp.zeros((pad_e,), jnp.int32)])
    dump = n + jnp.arange(pad_e, dtype=jnp.int32) % (np_pad - n)
    dstp = jnp.concatenate([dst, dump])
    dst3 = dstp.reshape(NW, cpt, CHUNK)

    mesh = plsc.VectorSubcoreMesh(core_axis_name="c", subcore_axis_name="s")

    deg_part = pl.kernel(
        functools.partial(_deg_body, np_pad, cpt),
        out_type=jax.ShapeDtypeStruct((NC, np_pad), jnp.float32),
        mesh=mesh,
        scratch_types=[
            pltpu.VMEM((cpt, CHUNK), jnp.int32),
            pltpu.VMEM((CHUNK,), jnp.float32),
            pltpu.VMEM((np_pad // NS,), jnp.float32),
            pltpu.VMEM_SHARED((np_pad,), jnp.float32),
        ],
    )(dst3)

    degt = deg_part.T
    hp, dis = pl.pallas_call(
        _prep_body,
        grid=(nblk,),
        in_specs=[
            pl.BlockSpec((rblk, d), lambda i: (i, 0)),
            pl.BlockSpec((d, h), lambda i: (0, 0)),
            pl.BlockSpec((rblk, NC), lambda i: (i, 0)),
        ],
        out_specs=[
            pl.BlockSpec((rblk, h), lambda i: (i, 0)),
            pl.BlockSpec((rblk, 1), lambda i: (i, 0)),
        ],
        out_shape=[
            jax.ShapeDtypeStruct((n, h), jnp.float32),
            jax.ShapeDtypeStruct((n, 1), jnp.float32),
        ],
    )(x, W1, degt)

    tot_chunks = ep // CHUNK
    cpt_pair = tot_chunks // NS
    cpt1 = max(8, int(round(cpt_pair * 0.35 / 8)) * 8)
    cpt0 = cpt_pair - cpt1
    cptmax = max(cpt0, cpt1)
    src2 = srcp.reshape(tot_chunks, CHUNK)
    dst2 = dstp.reshape(tot_chunks, CHUNK)
    acc_part = pl.kernel(
        functools.partial(_seg_body, np_pad, cpt0, cpt1),
        out_type=jax.ShapeDtypeStruct((NC, np_pad, h), jnp.float32),
        mesh=mesh,
        scratch_types=[
            pltpu.VMEM((cptmax, CHUNK), jnp.int32),
            pltpu.VMEM((cptmax, CHUNK), jnp.int32),
            pltpu.VMEM((CHUNK, h), jnp.float32),
            pltpu.VMEM_SHARED((np_pad, h), jnp.float32),
            pltpu.SemaphoreType.DMA,
        ],
    )(hp, src2, dst2)

    out = pl.pallas_call(
        _out_body,
        grid=(nblk,),
        in_specs=[
            pl.BlockSpec((NC, rblk, h), lambda i: (0, i, 0)),
            pl.BlockSpec((rblk, h), lambda i: (i, 0)),
            pl.BlockSpec((rblk, 1), lambda i: (i, 0)),
            pl.BlockSpec((1, h), lambda i: (0, 0)),
            pl.BlockSpec((cls, h), lambda i: (0, 0)),
            pl.BlockSpec((cls, h), lambda i: (0, 0)),
            pl.BlockSpec((cls, h), lambda i: (0, 0)),
            pl.BlockSpec((1, cls), lambda i: (0, 0)),
            pl.BlockSpec((1, cls), lambda i: (0, 0)),
            pl.BlockSpec((1, cls), lambda i: (0, 0)),
        ],
        out_specs=pl.BlockSpec((rblk, cls), lambda i: (i, 0)),
        out_shape=jax.ShapeDtypeStruct((n, cls), jnp.float32),
    )(acc_part, hp, dis, b1.reshape(1, h), w_mu, w_log_sigma, eps_w,
      b_mu.reshape(1, cls), b_log_sigma.reshape(1, cls),
      eps_b.reshape(1, cls))

    return out


# speedup vs baseline: 1.7210x; 1.4908x over previous
import functools

import jax
import jax.numpy as jnp
from jax import lax
from jax.experimental import pallas as pl
from jax.experimental.pallas import tpu as pltpu
from jax.experimental.pallas import tpu_sc as plsc

L = 16
NC = 2
NS = 16
NW = NC * NS
CHUNK = 128


def _cdiv(a, b):
    return (a + b - 1) // b


def _deg_body(np_pad, cpt, dst_hbm, deg_out, dst_v, ones_v, z_v, deg_sh):
    c = lax.axis_index("c")
    s = lax.axis_index("s")
    wid = c * NS + s
    rpt = np_pad // NS

    def zrow(i, _):
        z_v[pl.ds(i * L, L)] = jnp.zeros((L,), jnp.float32)
        return 0
    lax.fori_loop(0, rpt // L, zrow, 0)
    for k in range(CHUNK // L):
        ones_v[pl.ds(k * L, L)] = jnp.ones((L,), jnp.float32)

    pltpu.sync_copy(z_v, deg_sh.at[pl.ds(s * rpt, rpt)])
    plsc.subcore_barrier()

    pltpu.sync_copy(dst_hbm.at[wid], dst_v)

    def body(i, _):
        pltpu.sync_copy(ones_v, deg_sh.at[dst_v.at[i]], add=True)
        return 0
    lax.fori_loop(0, cpt, body, 0)

    plsc.subcore_barrier()
    pltpu.sync_copy(deg_sh.at[pl.ds(s * rpt, rpt)],
                    deg_out.at[c, pl.ds(s * rpt, rpt)])


def _seg_body(np_pad, cpt, hp_hbm, src_hbm, dst_hbm, acc_out,
              src_v, dst_v, rows_v, acc_sh, sem):
    c = lax.axis_index("c")
    s = lax.axis_index("s")
    wid = c * NS + s
    rpt = np_pad // NS

    def zrow(i, _):
        for k in range(8):
            rows_v[i, pl.ds(k * L, L)] = jnp.zeros((L,), jnp.float32)
        return 0
    lax.fori_loop(0, CHUNK, zrow, 0)

    def zcopy(j, _):
        pltpu.sync_copy(rows_v, acc_sh.at[pl.ds(s * rpt + j * CHUNK, CHUNK), :])
        return 0
    lax.fori_loop(0, rpt // CHUNK, zcopy, 0)
    plsc.subcore_barrier()

    pltpu.sync_copy(src_hbm.at[wid], src_v)
    pltpu.sync_copy(dst_hbm.at[wid], dst_v)

    def body(i, _):
        pltpu.async_copy(hp_hbm.at[src_v.at[i]], rows_v, sem).wait()
        pltpu.sync_copy(rows_v, acc_sh.at[dst_v.at[i]], add=True)
        return 0
    lax.fori_loop(0, cpt, body, 0)

    plsc.subcore_barrier()
    pltpu.sync_copy(acc_sh.at[pl.ds(s * rpt, rpt), :],
                    acc_out.at[c, pl.ds(s * rpt, rpt), :])


def _prep_body(x_ref, w1_ref, degt_ref, hp_ref, dis_ref):
    deg = degt_ref[:, 0:1] + degt_ref[:, 1:2] + 1.0
    dis = lax.rsqrt(jnp.maximum(deg, 1e-12))
    h = jnp.dot(x_ref[...], w1_ref[...], preferred_element_type=jnp.float32)
    hp_ref[...] = h * dis
    dis_ref[...] = dis


def _out_body(acc_ref, hp_ref, dis_ref, b1_ref, wmu_ref, wls_ref, epsw_ref,
              bmu_ref, bls_ref, epsb_ref, out_ref):
    t = acc_ref[0] + acc_ref[1] + hp_ref[...]
    pre = t * dis_ref[...] + b1_ref[...]
    hr = jnp.maximum(pre, 0.0)
    w = wmu_ref[...] + jnp.exp(wls_ref[...]) * epsw_ref[...]
    b = bmu_ref[...] + jnp.exp(bls_ref[...]) * epsb_ref[...]
    logits = lax.dot_general(hr, w, (((1,), (1,)), ((), ())),
                             preferred_element_type=jnp.float32) + b
    m = jnp.max(logits, axis=1, keepdims=True)
    ex = jnp.exp(logits - m)
    lse = m + jnp.log(jnp.sum(ex, axis=1, keepdims=True))
    out_ref[...] = logits - lse


def kernel(x, edge_index, W1, b1, w_mu, w_log_sigma, eps_w, b_mu, b_log_sigma,
           eps_b):
    n, d = x.shape
    h = W1.shape[1]
    cls = w_mu.shape[0]
    e = edge_index.shape[1]

    np_pad = _cdiv(n + 1, NS * L) * NS * L
    ep = _cdiv(e, NW * CHUNK) * NW * CHUNK
    cpt = ep // (NW * CHUNK)
    rblk = 1024
    nblk = np_pad // rblk if np_pad % rblk == 0 else _cdiv(np_pad, rblk)
    rblk = np_pad // nblk
    assert np_pad % nblk == 0 and rblk % 8 == 0

    src = edge_index[0]
    dst = edge_index[1]
    pad_e = ep - e
    srcp = jnp.concatenate([src, jnp.zeros((pad_e,), jnp.int32)])
    dstp = jnp.concatenate([dst, jnp.full((pad_e,), n, jnp.int32)])
    src3 = srcp.reshape(NW, cpt, CHUNK)
    dst3 = dstp.reshape(NW, cpt, CHUNK)
    x_pad = jnp.concatenate([x, jnp.zeros((np_pad - n, d), jnp.float32)])

    mesh = plsc.VectorSubcoreMesh(core_axis_name="c", subcore_axis_name="s")

    deg_part = pl.kernel(
        functools.partial(_deg_body, np_pad, cpt),
        out_type=jax.ShapeDtypeStruct((NC, np_pad), jnp.float32),
        mesh=mesh,
        scratch_types=[
            pltpu.VMEM((cpt, CHUNK), jnp.int32),
            pltpu.VMEM((CHUNK,), jnp.float32),
            pltpu.VMEM((np_pad // NS,), jnp.float32),
            pltpu.VMEM_SHARED((np_pad,), jnp.float32),
        ],
    )(dst3)

    degt = deg_part.T
    hp, dis = pl.pallas_call(
        _prep_body,
        grid=(nblk,),
        in_specs=[
            pl.BlockSpec((rblk, d), lambda i: (i, 0)),
            pl.BlockSpec((d, h), lambda i: (0, 0)),
            pl.BlockSpec((rblk, NC), lambda i: (i, 0)),
        ],
        out_specs=[
            pl.BlockSpec((rblk, h), lambda i: (i, 0)),
            pl.BlockSpec((rblk, 1), lambda i: (i, 0)),
        ],
        out_shape=[
            jax.ShapeDtypeStruct((np_pad, h), jnp.float32),
            jax.ShapeDtypeStruct((np_pad, 1), jnp.float32),
        ],
    )(x_pad, W1, degt)

    acc_part = pl.kernel(
        functools.partial(_seg_body, np_pad, cpt),
        out_type=jax.ShapeDtypeStruct((NC, np_pad, h), jnp.float32),
        mesh=mesh,
        scratch_types=[
            pltpu.VMEM((cpt, CHUNK), jnp.int32),
            pltpu.VMEM((cpt, CHUNK), jnp.int32),
            pltpu.VMEM((CHUNK, h), jnp.float32),
            pltpu.VMEM_SHARED((np_pad, h), jnp.float32),
            pltpu.SemaphoreType.DMA,
        ],
    )(hp, src3, dst3)

    out_pad = pl.pallas_call(
        _out_body,
        grid=(nblk,),
        in_specs=[
            pl.BlockSpec((NC, rblk, h), lambda i: (0, i, 0)),
            pl.BlockSpec((rblk, h), lambda i: (i, 0)),
            pl.BlockSpec((rblk, 1), lambda i: (i, 0)),
            pl.BlockSpec((1, h), lambda i: (0, 0)),
            pl.BlockSpec((cls, h), lambda i: (0, 0)),
            pl.BlockSpec((cls, h), lambda i: (0, 0)),
            pl.BlockSpec((cls, h), lambda i: (0, 0)),
            pl.BlockSpec((1, cls), lambda i: (0, 0)),
            pl.BlockSpec((1, cls), lambda i: (0, 0)),
            pl.BlockSpec((1, cls), lambda i: (0, 0)),
        ],
        out_specs=pl.BlockSpec((rblk, cls), lambda i: (i, 0)),
        out_shape=jax.ShapeDtypeStruct((np_pad, cls), jnp.float32),
    )(acc_part, hp, dis, b1.reshape(1, h), w_mu, w_log_sigma, eps_w,
      b_mu.reshape(1, cls), b_log_sigma.reshape(1, cls),
      eps_b.reshape(1, cls))

    return out_pad[:n]
